# baseline (speedup 1.0000x reference)
import functools

import jax
import jax.numpy as jnp
from jax import lax
from jax.experimental import pallas as pl
from jax.experimental.pallas import tpu as pltpu
from jax.experimental.pallas import tpu_sc as plsc

VOCAB_N = 1000000
EMBED_N = 64
BATCH_N = 16384

_NC = 2
_NS = 16
_RT = 1024
_NQ = (VOCAB_N + _RT - 1) // _RT
_TAIL_Q = _NQ - 1
_TAIL_ROWS = VOCAB_N - _TAIL_Q * _RT
_KH = 4
_COMP = _KH * 8
_QCAP = 512
_NRING = (_QCAP // 16 + 1) * 4
_IDXBLK = 2048
_HALF = BATCH_N * _COMP
_SAC = _NC * _HALF
_OUT_N = _SAC + 512

_mesh = plsc.VectorSubcoreMesh(core_axis_name="c", subcore_axis_name="s")


@functools.partial(
    pl.kernel,
    mesh=_mesh,
    out_type=jax.ShapeDtypeStruct((_OUT_N,), jnp.float32),
    scratch_types=[
        pltpu.VMEM((_COMP, _RT), jnp.float32),
        pltpu.VMEM((_IDXBLK,), jnp.int32),
        pltpu.VMEM((BATCH_N + 16,), jnp.int32),
        pltpu.VMEM((BATCH_N + 16,), jnp.int32),
        pltpu.VMEM((_QCAP + 16,), jnp.int32),
        pltpu.VMEM((_QCAP + 16,), jnp.int32),
        pltpu.VMEM((_NRING, 128), jnp.float32),
        pltpu.VMEM((_NRING, 128), jnp.int32),
        pltpu.VMEM((512,), jnp.float32),
        pltpu.SemaphoreType.DMA,
        pltpu.SemaphoreType.DMA,
    ],
    compiler_params=pltpu.CompilerParams(
        use_tc_tiling_on_sc=True, needs_layout_passes=False
    ),
)
def _lookup(tT_hbm, tail_hbm, idx_hbm, out_hbm, buf_v, iblk_v, rlist_v,
            ilist_v, rq_v, iq_v, blk_v, widx_v, dummy_v, sem, ssem):
    cid = lax.axis_index("c")
    sid = lax.axis_index("s")
    lanes = lax.iota(jnp.int32, 16)
    one = jnp.ones((16,), jnp.int32)
    zero = jnp.zeros((16,), jnp.int32)

    def p1_inner(j, carry, blk):
        cnt = carry
        v = iblk_v[pl.ds(j * 16, 16)]
        ivec = blk * _IDXBLK + j * 16 + lanes
        m = ((v >> 10) & 15) == sid
        mi = jnp.where(m, one, zero)
        cs = plsc.cumsum(mi)
        dst = jnp.where(m, cnt + cs - 1, BATCH_N + lanes)
        plsc.store_scatter(rlist_v, [dst], v)
        plsc.store_scatter(ilist_v, [dst], ivec)
        return cnt + jnp.sum(mi)

    def p1_outer(blk, cnt):
        pltpu.sync_copy(idx_hbm.at[pl.ds(blk * _IDXBLK, _IDXBLK)], iblk_v)
        return lax.fori_loop(
            0, _IDXBLK // 16, functools.partial(p1_inner, blk=blk), cnt
        )

    cnt = lax.fori_loop(0, BATCH_N // _IDXBLK, p1_outer, jnp.int32(0))

    sub0 = pl.multiple_of(cid * _COMP, 8)
    cid_off = cid * _HALF

    def fire_chunk(q):
        lane0 = pl.multiple_of(q * _RT, 128)

        @pl.when(q != _TAIL_Q)
        def _():
            for k in range(_KH):
                pltpu.async_copy(
                    tT_hbm.at[pl.ds(sub0 + k * 8, 8), pl.ds(lane0, _RT)],
                    buf_v.at[pl.ds(k * 8, 8), :],
                    sem,
                )

        @pl.when(q == _TAIL_Q)
        def _():
            for k in range(_KH):
                pltpu.async_copy(
                    tail_hbm.at[pl.ds(sub0 + k * 8, 8), :],
                    buf_v.at[pl.ds(k * 8, 8), :],
                    sem,
                )

    def wait_chunk(q):
        pltpu.make_async_copy(
            tT_hbm.at[pl.ds(0, _COMP), pl.ds(0, _RT)], buf_v, sem
        ).wait()

    def drain(n, q):
        def blkbody(b, carry):
            rq = rq_v[pl.ds(b * 16, 16)]
            iq = iq_v[pl.ds(b * 16, 16)]
            pos = b * 16 + lanes
            valid = pos < n
            rloc = jnp.clip(rq - q * _RT, 0, _RT - 1)
            base = jnp.where(
                valid, iq * _COMP + cid_off, _SAC + lanes * _COMP
            )
            for comp in range(_COMP):
                crow = jnp.full((16,), comp, jnp.int32)
                j = lanes * _COMP + comp
                vals = plsc.load_gather(buf_v, [crow, rloc])
                plsc.store_scatter(blk_v, [b * 4 + (j >> 7), j & 127], vals)
                plsc.store_scatter(
                    widx_v, [b * 4 + (j >> 7), j & 127], base + comp
                )
            for jj in range(4):
                row = b * 4 + jj
                pltpu.async_copy(
                    blk_v.at[row], out_hbm.at[widx_v.at[row]], ssem
                )
            return carry

        nblk = (n + 15) // 16
        lax.fori_loop(0, nblk, blkbody, 0)

        def waitbody(w, carry):
            pltpu.make_async_copy(
                out_hbm.at[pl.ds(0, 512)], dummy_v, ssem
            ).wait()
            return carry

        lax.fori_loop(0, nblk, waitbody, 0)

    my_nq = (_NQ - 1 - sid) // _NS + 1

    def round_body(g, carry):
        q = g * _NS + sid
        fire_chunk(q)
        wait_chunk(q)

        def scan_body(j, qcnt):
            rv = rlist_v[pl.ds(j * 16, 16)]
            iv = ilist_v[pl.ds(j * 16, 16)]
            pos = j * 16 + lanes
            m = ((rv >> 10) == q) & (pos < cnt)
            mi = jnp.where(m, one, zero)
            cs = plsc.cumsum(mi)
            dst = jnp.where(m, qcnt + cs - 1, _QCAP + lanes)
            plsc.store_scatter(rq_v, [dst], rv)
            plsc.store_scatter(iq_v, [dst], iv)
            qcnt2 = qcnt + jnp.sum(mi)
            full = qcnt2 >= _QCAP - 16

            @pl.when(full)
            def _():
                drain(qcnt2, q)

            return jnp.where(full, jnp.int32(0), qcnt2)

        qleft = lax.fori_loop(0, (cnt + 15) // 16, scan_body, jnp.int32(0))
        drain(qleft, q)
        return carry

    lax.fori_loop(0, my_nq, round_body, 0)


def kernel(agent_index, table):
    t_t = table.T
    tail = jnp.pad(
        t_t[:, _TAIL_Q * _RT :], ((0, 0), (0, _RT - _TAIL_ROWS))
    )
    o = _lookup(t_t, tail, agent_index.astype(jnp.int32))
    halves = o[:_SAC].reshape(_NC, BATCH_N, _COMP)
    return jnp.concatenate([halves[0], halves[1]], axis=1)


# speedup vs baseline: 83.1123x; 83.1123x over previous
import functools

import jax
import jax.numpy as jnp
from jax import lax
from jax.experimental import pallas as pl
from jax.experimental.pallas import tpu as pltpu
from jax.experimental.pallas import tpu_sc as plsc

VOCAB_N = 1000000
EMBED_N = 64
BATCH_N = 16384

_NC = 2
_NS = 16
_RT = 1024
_NQ = (VOCAB_N + _RT - 1) // _RT
_TAIL_Q = _NQ - 1
_TAIL_ROWS = VOCAB_N - _TAIL_Q * _RT
_KH = 4
_COMP = _KH * 8
_QCAP = 256
_NBLK = _QCAP // 16 + 1
_IDXBLK = 2048
_OUT_R = BATCH_N + 16

_mesh = plsc.VectorSubcoreMesh(core_axis_name="c", subcore_axis_name="s")


@functools.partial(
    pl.kernel,
    mesh=_mesh,
    out_type=(
        jax.ShapeDtypeStruct((_OUT_R, 128), jnp.float32),
        jax.ShapeDtypeStruct((_OUT_R, 128), jnp.float32),
    ),
    scratch_types=[
        pltpu.VMEM((_COMP, _RT), jnp.float32),
        pltpu.VMEM((_IDXBLK,), jnp.int32),
        pltpu.VMEM((BATCH_N + 16,), jnp.int32),
        pltpu.VMEM((BATCH_N + 16,), jnp.int32),
        pltpu.VMEM((_QCAP + 16,), jnp.int32),
        pltpu.VMEM((_QCAP + 16,), jnp.int32),
        pltpu.VMEM((_NBLK * 16, 128), jnp.float32),
        pltpu.VMEM((_NBLK, 16), jnp.int32),
        pltpu.VMEM((16, 128), jnp.float32),
        pltpu.SemaphoreType.DMA,
        pltpu.SemaphoreType.DMA,
    ],
    compiler_params=pltpu.CompilerParams(
        use_tc_tiling_on_sc=True, needs_layout_passes=False
    ),
)
def _lookup(tT_hbm, tail_hbm, idx_hbm, out0_hbm, out1_hbm, buf_v, iblk_v,
            rlist_v, ilist_v, rq_v, iq_v, blk_v, widx_v, dummy_v, sem, ssem):
    cid = lax.axis_index("c")
    sid = lax.axis_index("s")
    lanes = lax.iota(jnp.int32, 16)
    one = jnp.ones((16,), jnp.int32)
    zero = jnp.zeros((16,), jnp.int32)

    def p1_inner(j, carry, blk):
        cnt = carry
        v = iblk_v[pl.ds(j * 16, 16)]
        ivec = blk * _IDXBLK + j * 16 + lanes
        m = ((v >> 10) & 15) == sid
        mi = jnp.where(m, one, zero)
        cs = plsc.cumsum(mi)
        dst = jnp.where(m, cnt + cs - 1, BATCH_N + lanes)
        plsc.store_scatter(rlist_v, [dst], v)
        plsc.store_scatter(ilist_v, [dst], ivec)
        return cnt + jnp.sum(mi)

    def p1_outer(blk, cnt):
        pltpu.sync_copy(idx_hbm.at[pl.ds(blk * _IDXBLK, _IDXBLK)], iblk_v)
        return lax.fori_loop(
            0, _IDXBLK // 16, functools.partial(p1_inner, blk=blk), cnt
        )

    cnt = lax.fori_loop(0, BATCH_N // _IDXBLK, p1_outer, jnp.int32(0))

    sub0 = pl.multiple_of(cid * _COMP, 8)

    def fire_chunk(q):
        lane0 = pl.multiple_of(q * _RT, 128)

        @pl.when(q != _TAIL_Q)
        def _():
            for k in range(_KH):
                pltpu.async_copy(
                    tT_hbm.at[pl.ds(sub0 + k * 8, 8), pl.ds(lane0, _RT)],
                    buf_v.at[pl.ds(k * 8, 8), :],
                    sem,
                )

        @pl.when(q == _TAIL_Q)
        def _():
            for k in range(_KH):
                pltpu.async_copy(
                    tail_hbm.at[pl.ds(sub0 + k * 8, 8), :],
                    buf_v.at[pl.ds(k * 8, 8), :],
                    sem,
                )

    def wait_chunk(q):
        pltpu.make_async_copy(
            tT_hbm.at[pl.ds(0, _COMP), pl.ds(0, _RT)], buf_v, sem
        ).wait()

    def drain(n, q):
        def blkbody(b, carry):
            rq = rq_v[pl.ds(b * 16, 16)]
            iq = iq_v[pl.ds(b * 16, 16)]
            pos = b * 16 + lanes
            valid = pos < n
            rloc = jnp.clip(rq - q * _RT, 0, _RT - 1)
            rowid = jnp.where(valid, iq, BATCH_N + lanes)
            plsc.store_scatter(widx_v, [jnp.full((16,), b, jnp.int32),
                                        lanes], rowid)
            for comp in range(_COMP):
                crow = jnp.full((16,), comp, jnp.int32)
                vals = plsc.load_gather(buf_v, [crow, rloc])
                plsc.store_scatter(
                    blk_v, [b * 16 + lanes, crow], vals
                )

            @pl.when(cid == 0)
            def _():
                pltpu.async_copy(
                    blk_v.at[pl.ds(b * 16, 16), :],
                    out0_hbm.at[widx_v.at[b]],
                    ssem,
                )

            @pl.when(cid == 1)
            def _():
                pltpu.async_copy(
                    blk_v.at[pl.ds(b * 16, 16), :],
                    out1_hbm.at[widx_v.at[b]],
                    ssem,
                )

            return carry

        nblk = (n + 15) // 16
        lax.fori_loop(0, nblk, blkbody, 0)

        def waitbody(w, carry):
            pltpu.make_async_copy(
                tT_hbm.at[pl.ds(0, 16), pl.ds(0, 128)], dummy_v, ssem
            ).wait()
            return carry

        lax.fori_loop(0, nblk, waitbody, 0)

    my_nq = (_NQ - 1 - sid) // _NS + 1

    def round_body(g, carry):
        q = g * _NS + sid
        fire_chunk(q)
        wait_chunk(q)

        def scan_body(j, qcnt):
            rv = rlist_v[pl.ds(j * 16, 16)]
            iv = ilist_v[pl.ds(j * 16, 16)]
            pos = j * 16 + lanes
            m = ((rv >> 10) == q) & (pos < cnt)
            mi = jnp.where(m, one, zero)
            cs = plsc.cumsum(mi)
            dst = jnp.where(m, qcnt + cs - 1, _QCAP + lanes)
            plsc.store_scatter(rq_v, [dst], rv)
            plsc.store_scatter(iq_v, [dst], iv)
            qcnt2 = qcnt + jnp.sum(mi)
            full = qcnt2 >= _QCAP - 16

            @pl.when(full)
            def _():
                drain(qcnt2, q)

            return jnp.where(full, jnp.int32(0), qcnt2)

        qleft = lax.fori_loop(0, (cnt + 15) // 16, scan_body, jnp.int32(0))
        drain(qleft, q)
        return carry

    lax.fori_loop(0, my_nq, round_body, 0)


def kernel(agent_index, table):
    t_t = table.T
    tail = jnp.pad(
        t_t[:, _TAIL_Q * _RT :], ((0, 0), (0, _RT - _TAIL_ROWS))
    )
    o0, o1 = _lookup(t_t, tail, agent_index.astype(jnp.int32))
    return jnp.concatenate(
        [o0[:BATCH_N, :_COMP], o1[:BATCH_N, :_COMP]], axis=1
    )


# speedup vs baseline: 83.2117x; 1.0012x over previous
import functools

import jax
import jax.numpy as jnp
from jax import lax
from jax.experimental import pallas as pl
from jax.experimental.pallas import tpu as pltpu
from jax.experimental.pallas import tpu_sc as plsc

VOCAB_N = 1000000
EMBED_N = 64
BATCH_N = 16384

_NC = 2
_NS = 16
_RT = 1024
_NQ = (VOCAB_N + _RT - 1) // _RT
_TAIL_Q = _NQ - 1
_TAIL_ROWS = VOCAB_N - _TAIL_Q * _RT
_KH = 4
_COMP = _KH * 8
_QCAP = 256
_NBLK = _QCAP // 16 + 1
_IDXBLK = 2048
_OUT_R = BATCH_N + 16

_mesh = plsc.VectorSubcoreMesh(core_axis_name="c", subcore_axis_name="s")


@functools.partial(
    pl.kernel,
    mesh=_mesh,
    out_type=(
        jax.ShapeDtypeStruct((_OUT_R, 128), jnp.float32),
        jax.ShapeDtypeStruct((_OUT_R, 128), jnp.float32),
    ),
    scratch_types=[
        pltpu.VMEM((_COMP, _RT), jnp.float32),
        pltpu.VMEM((_IDXBLK,), jnp.int32),
        pltpu.VMEM((BATCH_N + 16,), jnp.int32),
        pltpu.VMEM((BATCH_N + 16,), jnp.int32),
        pltpu.VMEM((_QCAP + 16,), jnp.int32),
        pltpu.VMEM((_QCAP + 16,), jnp.int32),
        pltpu.VMEM((_NBLK * 16, 128), jnp.float32),
        pltpu.VMEM((_NBLK, 16), jnp.int32),
        pltpu.VMEM((16, 128), jnp.float32),
        pltpu.SemaphoreType.DMA,
        pltpu.SemaphoreType.DMA,
    ],
    compiler_params=pltpu.CompilerParams(
        use_tc_tiling_on_sc=True,
        needs_layout_passes=False,
        disable_bounds_checks=True,
        disable_semaphore_checks=True,
    ),
)
def _lookup(tT_hbm, tail_hbm, idx_hbm, out0_hbm, out1_hbm, buf_v, iblk_v,
            rlist_v, ilist_v, rq_v, iq_v, blk_v, widx_v, dummy_v, sem, ssem):
    cid = lax.axis_index("c")
    sid = lax.axis_index("s")
    lanes = lax.iota(jnp.int32, 16)
    one = jnp.ones((16,), jnp.int32)
    zero = jnp.zeros((16,), jnp.int32)

    def p1_inner(j, carry, blk):
        cnt = carry
        v = iblk_v[pl.ds(j * 16, 16)]
        ivec = blk * _IDXBLK + j * 16 + lanes
        m = ((v >> 10) & 15) == sid
        mi = jnp.where(m, one, zero)
        cs = plsc.cumsum(mi)
        dst = jnp.where(m, cnt + cs - 1, BATCH_N + lanes)
        plsc.store_scatter(rlist_v, [dst], v)
        plsc.store_scatter(ilist_v, [dst], ivec)
        return cnt + jnp.sum(mi)

    def p1_outer(blk, cnt):
        pltpu.sync_copy(idx_hbm.at[pl.ds(blk * _IDXBLK, _IDXBLK)], iblk_v)
        return lax.fori_loop(
            0, _IDXBLK // 16, functools.partial(p1_inner, blk=blk), cnt
        )

    cnt = lax.fori_loop(0, BATCH_N // _IDXBLK, p1_outer, jnp.int32(0))

    sub0 = pl.multiple_of(cid * _COMP, 8)

    def fire_chunk(q):
        lane0 = pl.multiple_of(q * _RT, 128)

        @pl.when(q != _TAIL_Q)
        def _():
            for k in range(_KH):
                pltpu.async_copy(
                    tT_hbm.at[pl.ds(sub0 + k * 8, 8), pl.ds(lane0, _RT)],
                    buf_v.at[pl.ds(k * 8, 8), :],
                    sem,
                )

        @pl.when(q == _TAIL_Q)
        def _():
            for k in range(_KH):
                pltpu.async_copy(
                    tail_hbm.at[pl.ds(sub0 + k * 8, 8), :],
                    buf_v.at[pl.ds(k * 8, 8), :],
                    sem,
                )

    def wait_chunk(q):
        pltpu.make_async_copy(
            tT_hbm.at[pl.ds(0, _COMP), pl.ds(0, _RT)], buf_v, sem
        ).wait()

    def drain(n, q):
        def blkbody(b, carry):
            rq = rq_v[pl.ds(b * 16, 16)]
            iq = iq_v[pl.ds(b * 16, 16)]
            pos = b * 16 + lanes
            valid = pos < n
            rloc = jnp.clip(rq - q * _RT, 0, _RT - 1)
            rowid = jnp.where(valid, iq, BATCH_N + lanes)
            plsc.store_scatter(widx_v, [jnp.full((16,), b, jnp.int32),
                                        lanes], rowid)
            for comp in range(_COMP):
                crow = jnp.full((16,), comp, jnp.int32)
                vals = plsc.load_gather(buf_v, [crow, rloc])
                plsc.store_scatter(
                    blk_v, [b * 16 + lanes, crow], vals
                )

            @pl.when(cid == 0)
            def _():
                pltpu.async_copy(
                    blk_v.at[pl.ds(b * 16, 16), :],
                    out0_hbm.at[widx_v.at[b]],
                    ssem,
                )

            @pl.when(cid == 1)
            def _():
                pltpu.async_copy(
                    blk_v.at[pl.ds(b * 16, 16), :],
                    out1_hbm.at[widx_v.at[b]],
                    ssem,
                )

            return carry

        nblk = (n + 15) // 16
        lax.fori_loop(0, nblk, blkbody, 0)

        def waitbody(w, carry):
            pltpu.make_async_copy(
                tT_hbm.at[pl.ds(0, 16), pl.ds(0, 128)], dummy_v, ssem
            ).wait()
            return carry

        lax.fori_loop(0, nblk, waitbody, 0)

    my_nq = (_NQ - 1 - sid) // _NS + 1

    def round_body(g, carry):
        q = g * _NS + sid
        fire_chunk(q)
        wait_chunk(q)

        def scan_body(j, qcnt):
            rv = rlist_v[pl.ds(j * 16, 16)]
            iv = ilist_v[pl.ds(j * 16, 16)]
            pos = j * 16 + lanes
            m = ((rv >> 10) == q) & (pos < cnt)
            mi = jnp.where(m, one, zero)
            cs = plsc.cumsum(mi)
            dst = jnp.where(m, qcnt + cs - 1, _QCAP + lanes)
            plsc.store_scatter(rq_v, [dst], rv)
            plsc.store_scatter(iq_v, [dst], iv)
            qcnt2 = qcnt + jnp.sum(mi)
            full = qcnt2 >= _QCAP - 16

            @pl.when(full)
            def _():
                drain(qcnt2, q)

            return jnp.where(full, jnp.int32(0), qcnt2)

        qleft = lax.fori_loop(0, (cnt + 15) // 16, scan_body, jnp.int32(0))
        drain(qleft, q)
        return carry

    lax.fori_loop(0, my_nq, round_body, 0)


def kernel(agent_index, table):
    t_t = table.T
    tail = jnp.pad(
        t_t[:, _TAIL_Q * _RT :], ((0, 0), (0, _RT - _TAIL_ROWS))
    )
    o0, o1 = _lookup(t_t, tail, agent_index.astype(jnp.int32))
    return jnp.concatenate(
        [o0[:BATCH_N, :_COMP], o1[:BATCH_N, :_COMP]], axis=1
    )


# speedup vs baseline: 113.9423x; 1.3693x over previous
import functools

import jax
import jax.numpy as jnp
from jax import lax
from jax.experimental import pallas as pl
from jax.experimental.pallas import tpu as pltpu
from jax.experimental.pallas import tpu_sc as plsc

VOCAB_N = 1000000
EMBED_N = 64
BATCH_N = 16384

_NC = 2
_NS = 16
_RT = 2048
_RTSH = 11
_NQ = (VOCAB_N + _RT - 1) // _RT
_TAIL_Q = _NQ - 1
_TAIL_ROWS = VOCAB_N - _TAIL_Q * _RT
_KH = 4
_COMP = _KH * 8
_QCAP = 128
_NBLK = _QCAP // 16 + 1
_IDXBLK = 2048
_OUT_R = BATCH_N + 16

_mesh = plsc.VectorSubcoreMesh(core_axis_name="c", subcore_axis_name="s")


@functools.partial(
    pl.kernel,
    mesh=_mesh,
    out_type=(
        jax.ShapeDtypeStruct((_OUT_R, 128), jnp.float32),
        jax.ShapeDtypeStruct((_OUT_R, 128), jnp.float32),
    ),
    scratch_types=[
        pltpu.VMEM((_COMP, _RT), jnp.float32),
        pltpu.VMEM((_IDXBLK,), jnp.int32),
        pltpu.VMEM((BATCH_N + 16,), jnp.int32),
        pltpu.VMEM((BATCH_N + 16,), jnp.int32),
        pltpu.VMEM((_QCAP + 16,), jnp.int32),
        pltpu.VMEM((_QCAP + 16,), jnp.int32),
        pltpu.VMEM((_NBLK * 16, 128), jnp.float32),
        pltpu.VMEM((_NBLK, 16), jnp.int32),
        pltpu.VMEM((16, 128), jnp.float32),
        pltpu.SemaphoreType.DMA,
        pltpu.SemaphoreType.DMA,
    ],
    compiler_params=pltpu.CompilerParams(
        use_tc_tiling_on_sc=True,
        needs_layout_passes=False,
        disable_bounds_checks=True,
        disable_semaphore_checks=True,
    ),
)
def _lookup(tT_hbm, tail_hbm, idx_hbm, out0_hbm, out1_hbm, buf_v, iblk_v,
            rlist_v, ilist_v, rq_v, iq_v, blk_v, widx_v, dummy_v, sem, ssem):
    cid = lax.axis_index("c")
    sid = lax.axis_index("s")
    lanes = lax.iota(jnp.int32, 16)
    one = jnp.ones((16,), jnp.int32)
    zero = jnp.zeros((16,), jnp.int32)

    def p1_inner(j, carry, blk):
        cnt = carry
        v = iblk_v[pl.ds(j * 16, 16)]
        ivec = blk * _IDXBLK + j * 16 + lanes
        m = ((v >> _RTSH) & 15) == sid
        mi = jnp.where(m, one, zero)
        cs = plsc.cumsum(mi)
        dst = jnp.where(m, cnt + cs - 1, BATCH_N + lanes)
        plsc.store_scatter(rlist_v, [dst], v)
        plsc.store_scatter(ilist_v, [dst], ivec)
        return cnt + jnp.sum(mi)

    def p1_outer(blk, cnt):
        pltpu.sync_copy(idx_hbm.at[pl.ds(blk * _IDXBLK, _IDXBLK)], iblk_v)
        return lax.fori_loop(
            0, _IDXBLK // 16, functools.partial(p1_inner, blk=blk), cnt
        )

    cnt = lax.fori_loop(0, BATCH_N // _IDXBLK, p1_outer, jnp.int32(0))

    sub0 = pl.multiple_of(cid * _COMP, 8)

    def fire_chunk(q):
        lane0 = pl.multiple_of(q * _RT, 128)

        @pl.when(q != _TAIL_Q)
        def _():
            for k in range(_KH):
                pltpu.async_copy(
                    tT_hbm.at[pl.ds(sub0 + k * 8, 8), pl.ds(lane0, _RT)],
                    buf_v.at[pl.ds(k * 8, 8), :],
                    sem,
                )

        @pl.when(q == _TAIL_Q)
        def _():
            for k in range(_KH):
                pltpu.async_copy(
                    tail_hbm.at[pl.ds(sub0 + k * 8, 8), :],
                    buf_v.at[pl.ds(k * 8, 8), :],
                    sem,
                )

    def wait_chunk(q):
        pltpu.make_async_copy(
            tT_hbm.at[pl.ds(0, _COMP), pl.ds(0, _RT)], buf_v, sem
        ).wait()

    def drain(n, q):
        def blkbody(b, carry):
            rq = rq_v[pl.ds(b * 16, 16)]
            iq = iq_v[pl.ds(b * 16, 16)]
            pos = b * 16 + lanes
            valid = pos < n
            rloc = jnp.clip(rq - q * _RT, 0, _RT - 1)
            rowid = jnp.where(valid, iq, BATCH_N + lanes)
            plsc.store_scatter(widx_v, [jnp.full((16,), b, jnp.int32),
                                        lanes], rowid)
            for comp in range(_COMP):
                crow = jnp.full((16,), comp, jnp.int32)
                vals = plsc.load_gather(buf_v, [crow, rloc])
                plsc.store_scatter(
                    blk_v, [b * 16 + lanes, crow], vals
                )

            @pl.when(cid == 0)
            def _():
                pltpu.async_copy(
                    blk_v.at[pl.ds(b * 16, 16), :],
                    out0_hbm.at[widx_v.at[b]],
                    ssem,
                )

            @pl.when(cid == 1)
            def _():
                pltpu.async_copy(
                    blk_v.at[pl.ds(b * 16, 16), :],
                    out1_hbm.at[widx_v.at[b]],
                    ssem,
                )

            return carry

        nblk = (n + 15) // 16
        lax.fori_loop(0, nblk, blkbody, 0)

        def waitbody(w, carry):
            pltpu.make_async_copy(
                tT_hbm.at[pl.ds(0, 16), pl.ds(0, 128)], dummy_v, ssem
            ).wait()
            return carry

        lax.fori_loop(0, nblk, waitbody, 0)

    my_nq = (_NQ - 1 - sid) // _NS + 1

    def round_body(g, carry):
        q = g * _NS + sid
        fire_chunk(q)
        wait_chunk(q)

        def scan_body(j, qcnt):
            rv = rlist_v[pl.ds(j * 16, 16)]
            iv = ilist_v[pl.ds(j * 16, 16)]
            pos = j * 16 + lanes
            m = ((rv >> _RTSH) == q) & (pos < cnt)
            mi = jnp.where(m, one, zero)
            cs = plsc.cumsum(mi)
            dst = jnp.where(m, qcnt + cs - 1, _QCAP + lanes)
            plsc.store_scatter(rq_v, [dst], rv)
            plsc.store_scatter(iq_v, [dst], iv)
            qcnt2 = qcnt + jnp.sum(mi)
            full = qcnt2 >= _QCAP - 16

            @pl.when(full)
            def _():
                drain(qcnt2, q)

            return jnp.where(full, jnp.int32(0), qcnt2)

        qleft = lax.fori_loop(0, (cnt + 15) // 16, scan_body, jnp.int32(0))
        drain(qleft, q)
        return carry

    lax.fori_loop(0, my_nq, round_body, 0)


def kernel(agent_index, table):
    t_t = table.T
    tail = jnp.pad(
        t_t[:, _TAIL_Q * _RT :], ((0, 0), (0, _RT - _TAIL_ROWS))
    )
    o0, o1 = _lookup(t_t, tail, agent_index.astype(jnp.int32))
    return jnp.concatenate(
        [o0[:BATCH_N, :_COMP], o1[:BATCH_N, :_COMP]], axis=1
    )
